# Initial kernel scaffold; baseline (speedup 1.0000x reference)
import jax
import jax.numpy as jnp
from jax.experimental import pallas as pl


def kernel(x, edge_index, batch, embed, conv_w1, conv_b1, bn1_g, bn1_b, conv_w2, conv_b2, bn2_g, bn2_b, lin1_w, lin1_b, lin2_w, lin2_b):
    raise NotImplementedError("write your pallas kernel here")


# speedup vs baseline: 7.5997x; 7.5997x over previous
import functools

import jax
import jax.numpy as jnp
from jax import lax
from jax.experimental import pallas as pl
from jax.experimental.pallas import tpu as pltpu
from jax.experimental.pallas import tpu_sc as plsc

N = 50000
E = 800000
A = 100
D = 64
B = 128
OUT = 10

NH = 32
NSC = 2
NTILE = 16
CH = 128
NCH = E // CH
NB = 25
BN_ROWS = N // NB
NPS = 50048
TROWS = NPS // NTILE
ZROWS = 136
_BN_SCALE = 1.0 / (1.0 + 1e-5) ** 0.5


def _argmax_rows(xb):
    mx = jnp.max(xb, axis=-1, keepdims=True)
    ii = lax.broadcasted_iota(jnp.int32, xb.shape, xb.ndim - 1)
    return jnp.min(jnp.where(xb == mx, ii, A), axis=-1).astype(jnp.int32)


def _stage_a_body(x_ref, emb_ref, typ_ref, r0_ref):
    i = pl.program_id(0)
    typ_ref[...] = _argmax_rows(x_ref[...].reshape(8, 250, A))

    @pl.when(i == 0)
    def _():
        e = emb_ref[...]
        m = jnp.maximum(2.0 * e[:, None, :] + e[None, :, :], 0.0)
        r0_ref[...] = jnp.stack(
            [m[:, :, :NH].reshape(A * A, NH), m[:, :, NH:].reshape(A * A, NH)], 0)


def _stage_a(x, embed):
    return pl.pallas_call(
        _stage_a_body,
        grid=(NB,),
        in_specs=[
            pl.BlockSpec((BN_ROWS, A), lambda i: (i, 0)),
            pl.BlockSpec((A, D), lambda i: (0, 0)),
        ],
        out_specs=[
            pl.BlockSpec((8, 250), lambda i: (i, 0)),
            pl.BlockSpec((NSC, A * A, NH), lambda i: (0, 0, 0)),
        ],
        out_shape=[
            jax.ShapeDtypeStruct((NB * 8, 250), jnp.int32),
            jax.ShapeDtypeStruct((NSC, A * A, NH), jnp.float32),
        ],
    )(x, embed)


def _edge_call(li, tbl_rows, K):
    mesh = plsc.VectorSubcoreMesh(core_axis_name="c", subcore_axis_name="s")

    scratch = [pltpu.VMEM_SHARED((NPS, NH), jnp.float32)]
    if li == 1:
        scratch += [pltpu.VMEM_SHARED((tbl_rows, NH), jnp.float32)]
    for _ in range(K):
        scratch += [pltpu.VMEM((CH,), jnp.int32),
                    pltpu.VMEM((CH,), jnp.int32),
                    pltpu.VMEM((CH,), jnp.int32),
                    pltpu.VMEM((CH,), jnp.int32),
                    pltpu.VMEM((CH,), jnp.int32),
                    pltpu.VMEM((CH, NH), jnp.float32)]
        if li == 1:
            scratch += [pltpu.VMEM((CH, NH), jnp.float32)]
    nsem = K * (5 if li == 1 else 4)
    scratch += [pltpu.SemaphoreType.DMA] * nsem

    def body(*refs):
        if li == 0:
            src_hbm, dst_hbm, typ_hbm, tbl_hbm = refs[:4]
            w_hbm = None
            agg_out = refs[4]
            rest = list(refs[5:])
        else:
            src_hbm, dst_hbm, typ_hbm, tbl_hbm, w_hbm = refs[:5]
            agg_out = refs[5]
            rest = list(refs[6:])
        agg_sp = rest.pop(0)
        tbl_sp = rest.pop(0) if li == 1 else None
        per = 7 if li == 1 else 6
        slots = [rest[k * per:(k + 1) * per] for k in range(K)]
        sems = rest[K * per:]
        sem_it = iter(sems)
        slot_sems = [[next(sem_it) for _ in range(5 if li == 1 else 4)]
                     for _ in range(K)]

        c = lax.axis_index("c")
        s = lax.axis_index("s")

        if li == 1:
            @pl.when(s == 0)
            def _():
                pltpu.sync_copy(tbl_hbm.at[pl.ds(c * tbl_rows, tbl_rows)],
                                tbl_sp)

        zsrc = slots[0][5]

        def _zfill(r, _):
            zsrc[r, pl.ds(0, 16)] = jnp.zeros((16,), jnp.float32)
            zsrc[r, pl.ds(16, 16)] = jnp.zeros((16,), jnp.float32)
            return _
        lax.fori_loop(0, CH, _zfill, None)

        def _zcopy(j, _):
            pltpu.sync_copy(zsrc, agg_sp.at[pl.ds(s * TROWS + j * CH, CH)])
            return _
        lax.fori_loop(0, TROWS // CH, _zcopy, None)
        pltpu.sync_copy(zsrc.at[pl.ds(0, TROWS % CH)],
                        agg_sp.at[pl.ds(s * TROWS + TROWS - TROWS % CH,
                                        TROWS % CH)])
        plsc.subcore_barrier()

        base = s * 390 + jnp.minimum(s, 10)
        n = jnp.where(s < 10, 391, 390)

        def _iter(i4, _):
            i0 = i4 * K
            valid = [i0 + j < n for j in range(K)]
            handles = [[None] * 3 for _ in range(K)]

            for j in range(K):
                sv, dv = slots[j][0], slots[j][1]
                ls, ld2 = slot_sems[j][0], slot_sems[j][1]
                off = (base + i0 + j) * CH

                @pl.when(valid[j])
                def _(j=j, sv=sv, dv=dv, ls=ls, ld2=ld2, off=off):
                    handles[j][0] = pltpu.async_copy(
                        src_hbm.at[pl.ds(off, CH)], sv, ls)
                    handles[j][1] = pltpu.async_copy(
                        dst_hbm.at[pl.ds(off, CH)], dv, ld2)

            th = [[None, None] for _ in range(K)]
            for j in range(K):
                sv, dv, ts, td = slots[j][:4]
                s2, s3 = slot_sems[j][2], slot_sems[j][3]

                @pl.when(valid[j])
                def _(j=j, sv=sv, dv=dv, ts=ts, td=td, s2=s2, s3=s3):
                    handles[j][0].wait()
                    handles[j][1].wait()
                    if li == 0:
                        th[j][0] = pltpu.async_copy(typ_hbm.at[sv], ts, s2)
                    th[j][1] = pltpu.async_copy(typ_hbm.at[dv], td, s3)

            gh = [[None, None] for _ in range(K)]
            for j in range(K):
                sv, dv, ts, td, gi, msg = slots[j][:6]
                s2, s3 = slot_sems[j][2], slot_sems[j][3]

                @pl.when(valid[j])
                def _(j=j, sv=sv, ts=ts, td=td, gi=gi, msg=msg, s2=s2, s3=s3):
                    if li == 0:
                        th[j][0].wait()
                        th[j][1].wait()
                        coff = jnp.broadcast_to(c * tbl_rows,
                                                (16,)).astype(jnp.int32)
                        for g in range(CH // 16):
                            sl = pl.ds(g * 16, 16)
                            gi[sl] = ts[sl] * A + td[sl] + coff
                        gh[j][0] = pltpu.async_copy(tbl_hbm.at[gi], msg, s2)
                    else:
                        th[j][1].wait()
                        coff = jnp.broadcast_to(c * N, (16,)).astype(jnp.int32)
                        for g in range(CH // 16):
                            sl = pl.ds(g * 16, 16)
                            gi[sl] = sv[sl] + coff
                        gh[j][0] = pltpu.async_copy(w_hbm.at[gi], slots[j][6], s2)
                        gh[j][1] = pltpu.async_copy(tbl_sp.at[td], msg, s3)

            for j in range(K):
                dv, msg = slots[j][1], slots[j][5]

                @pl.when(valid[j])
                def _(j=j, dv=dv, msg=msg):
                    gh[j][0].wait()
                    if li == 1:
                        gh[j][1].wait()
                        wr = slots[j][6]

                        def _relu_row(r, _):
                            for g in range(2):
                                sl = pl.ds(g * 16, 16)
                                msg[r, sl] = jnp.maximum(
                                    wr[r, sl] + msg[r, sl], 0.0)
                            return _
                        lax.fori_loop(0, CH, _relu_row, None)
                    pltpu.sync_copy(msg, agg_sp.at[dv], add=True)
            return _

        lax.fori_loop(0, (391 + K - 1) // K, _iter, None)
        plsc.subcore_barrier()

        def _drain(j, _):
            r = s * TROWS + j * ZROWS
            pltpu.sync_copy(agg_sp.at[pl.ds(r, ZROWS)],
                            agg_out.at[pl.ds(c * NPS + r, ZROWS)])
            return _
        lax.fori_loop(0, TROWS // ZROWS, _drain, None)

    out_type = jax.ShapeDtypeStruct((NSC * NPS, NH), jnp.float32)
    return functools.partial(
        pl.kernel, body, out_type=out_type, mesh=mesh, scratch_types=scratch,
        compiler_params=pltpu.CompilerParams(use_tc_tiling_on_sc=False))()


def _mlp(z, li, cw1, cb1, g1, b1, cw2, cb2, g2, b2):
    s1 = g1[li] * _BN_SCALE
    z = jnp.dot(z, cw1[li] * s1[None, :], precision=lax.Precision.HIGHEST)
    z = jnp.maximum(z + (cb1[li] * s1 + b1[li])[None, :], 0.0)
    s2 = g2[li] * _BN_SCALE
    z = jnp.dot(z, cw2[li] * s2[None, :], precision=lax.Precision.HIGHEST)
    return jnp.maximum(z + (cb2[li] * s2 + b2[li])[None, :], 0.0)


def _onehot_embed(typ2d, emb):
    oh = (typ2d[:, :, None] == lax.broadcasted_iota(
        jnp.int32, typ2d.shape + (A,), 2))
    return jnp.dot(oh.reshape(BN_ROWS, A).astype(jnp.float32), emb,
                   precision=lax.Precision.HIGHEST)


def _c0_body(typ_ref, agg_ref, emb_ref, cw1_ref, cb1_ref, g1_ref, b1_ref,
             cw2_ref, cb2_ref, g2_ref, b2_ref, w_ref):
    vx = _onehot_embed(typ_ref[...], emb_ref[...])
    agg = agg_ref[...]
    z = vx + jnp.concatenate([agg[0], agg[1]], axis=1)
    h = _mlp(z, 0, cw1_ref[...], cb1_ref[...], g1_ref[...], b1_ref[...],
             cw2_ref[...], cb2_ref[...], g2_ref[...], b2_ref[...])
    w = h + vx
    w_ref[...] = jnp.stack([w[:, :NH], w[:, NH:]], 0)


def _stage_c0(typ2d, agg, embed, cw1, cb1, g1, b1, cw2, cb2, g2, b2):
    wfull = pl.BlockSpec((NSC, D, D), lambda i: (0, 0, 0))
    vfull = pl.BlockSpec((NSC, D), lambda i: (0, 0))
    return pl.pallas_call(
        _c0_body,
        grid=(NB,),
        in_specs=[
            pl.BlockSpec((8, 250), lambda i: (i, 0)),
            pl.BlockSpec((NSC, BN_ROWS, NH), lambda i: (0, i, 0)),
            pl.BlockSpec((A, D), lambda i: (0, 0)),
            wfull, vfull, vfull, vfull, wfull, vfull, vfull, vfull,
        ],
        out_specs=pl.BlockSpec((NSC, BN_ROWS, NH), lambda i: (0, i, 0)),
        out_shape=jax.ShapeDtypeStruct((NSC, N, NH), jnp.float32),
    )(typ2d, agg, embed, cw1, cb1, g1, b1, cw2, cb2, g2, b2)


def _c1_body(typ_ref, w_ref, agg_ref, batch_ref, emb_ref, cw1_ref, cb1_ref,
             g1_ref, b1_ref, cw2_ref, cb2_ref, g2_ref, b2_ref, l1w_ref,
             l1b_ref, l2w_ref, l2b_ref, o_ref, pool_ref):
    i = pl.program_id(0)
    vx = _onehot_embed(typ_ref[...], emb_ref[...])
    w = w_ref[...]
    h1 = jnp.concatenate([w[0], w[1]], axis=1) - vx
    agg = agg_ref[...]
    z = h1 + jnp.concatenate([agg[0], agg[1]], axis=1)
    h2 = _mlp(z, 1, cw1_ref[...], cb1_ref[...], g1_ref[...], b1_ref[...],
              cw2_ref[...], cb2_ref[...], g2_ref[...], b2_ref[...])
    bt = batch_ref[...]
    oh = (bt[:, :, None] == lax.broadcasted_iota(
        jnp.int32, bt.shape + (B,), 2)).reshape(BN_ROWS, B)
    part = lax.dot_general(oh.astype(jnp.float32), h2,
                           (((0,), (0,)), ((), ())),
                           precision=lax.Precision.HIGHEST)

    @pl.when(i == 0)
    def _():
        pool_ref[...] = jnp.zeros_like(pool_ref)

    pool_ref[...] += part

    @pl.when(i == NB - 1)
    def _():
        p = pool_ref[...]
        o = jnp.maximum(jnp.dot(p, l1w_ref[...],
                                precision=lax.Precision.HIGHEST)
                        + l1b_ref[...], 0.0)
        o_ref[...] = jnp.dot(o, l2w_ref[...],
                             precision=lax.Precision.HIGHEST) + l2b_ref[...]


def _stage_c1(typ2d, w, agg, batch2d, embed, cw1, cb1, g1, b1, cw2, cb2, g2,
              b2, l1w, l1b, l2w, l2b):
    wfull = pl.BlockSpec((NSC, D, D), lambda i: (0, 0, 0))
    vfull = pl.BlockSpec((NSC, D), lambda i: (0, 0))
    return pl.pallas_call(
        _c1_body,
        grid=(NB,),
        in_specs=[
            pl.BlockSpec((8, 250), lambda i: (i, 0)),
            pl.BlockSpec((NSC, BN_ROWS, NH), lambda i: (0, i, 0)),
            pl.BlockSpec((NSC, BN_ROWS, NH), lambda i: (0, i, 0)),
            pl.BlockSpec((8, 250), lambda i: (i, 0)),
            pl.BlockSpec((A, D), lambda i: (0, 0)),
            wfull, vfull, vfull, vfull, wfull, vfull, vfull, vfull,
            pl.BlockSpec((D, D), lambda i: (0, 0)),
            pl.BlockSpec((1, D), lambda i: (0, 0)),
            pl.BlockSpec((D, OUT), lambda i: (0, 0)),
            pl.BlockSpec((1, OUT), lambda i: (0, 0)),
        ],
        out_specs=pl.BlockSpec((B, OUT), lambda i: (0, 0)),
        out_shape=jax.ShapeDtypeStruct((B, OUT), jnp.float32),
        scratch_shapes=[pltpu.VMEM((B, D), jnp.float32)],
    )(typ2d, w, agg, batch2d, embed, cw1, cb1, g1, b1, cw2, cb2, g2, b2,
      l1w, l1b, l2w, l2b)


def kernel(x, edge_index, batch, embed, conv_w1, conv_b1, bn1_g, bn1_b,
           conv_w2, conv_b2, bn2_g, bn2_b, lin1_w, lin1_b, lin2_w, lin2_b):
    typ2d, r0 = _stage_a(x, embed)
    typ = typ2d.reshape(N)
    src = edge_index[0]
    dst = edge_index[1]

    agg0 = _edge_call(0, A * A, 4)(src, dst, typ, r0.reshape(NSC * A * A, NH))
    w1 = _stage_c0(typ2d, agg0.reshape(NSC, NPS, NH), embed, conv_w1, conv_b1,
                   bn1_g, bn1_b, conv_w2, conv_b2, bn2_g, bn2_b)

    zpad = jnp.zeros((4, NH), jnp.float32)
    vt = jnp.concatenate([embed[:, :NH], zpad, embed[:, NH:], zpad], axis=0)
    agg1 = _edge_call(1, 104, 3)(src, dst, typ, vt, w1.reshape(NSC * N, NH))

    return _stage_c1(typ2d, w1, agg1.reshape(NSC, NPS, NH),
                     batch.reshape(NB * 8, 250), embed, conv_w1, conv_b1,
                     bn1_g, bn1_b, conv_w2, conv_b2, bn2_g, bn2_b,
                     lin1_w.reshape(D, D), lin1_b.reshape(1, D),
                     lin2_w.reshape(D, OUT), lin2_b.reshape(1, OUT))


# speedup vs baseline: 8.3084x; 1.0933x over previous
import functools

import jax
import jax.numpy as jnp
from jax import lax
from jax.experimental import pallas as pl
from jax.experimental.pallas import tpu as pltpu
from jax.experimental.pallas import tpu_sc as plsc

N = 50000
E = 800000
A = 100
D = 64
B = 128
OUT = 10

NH = 32
NSC = 2
NTILE = 16
CH = 128
NCH = E // CH
NB = 25
BN_ROWS = N // NB
NPS = 50048
TROWS = NPS // NTILE
ZROWS = 136
_BN_SCALE = 1.0 / (1.0 + 1e-5) ** 0.5


def _argmax_rows(xb):
    mx = jnp.max(xb, axis=-1, keepdims=True)
    ii = lax.broadcasted_iota(jnp.int32, xb.shape, xb.ndim - 1)
    return jnp.min(jnp.where(xb == mx, ii, A), axis=-1).astype(jnp.int32)


def _stage_a_body(x_ref, emb_ref, typ_ref, r0_ref):
    i = pl.program_id(0)
    typ_ref[...] = _argmax_rows(x_ref[...].reshape(8, 250, A))

    @pl.when(i == 0)
    def _():
        e = emb_ref[...]
        m = jnp.maximum(2.0 * e[:, None, :] + e[None, :, :], 0.0)
        r0_ref[...] = jnp.stack(
            [m[:, :, :NH].reshape(A * A, NH), m[:, :, NH:].reshape(A * A, NH)], 0)


def _stage_a(x, embed):
    return pl.pallas_call(
        _stage_a_body,
        grid=(NB,),
        in_specs=[
            pl.BlockSpec((BN_ROWS, A), lambda i: (i, 0)),
            pl.BlockSpec((A, D), lambda i: (0, 0)),
        ],
        out_specs=[
            pl.BlockSpec((8, 250), lambda i: (i, 0)),
            pl.BlockSpec((NSC, A * A, NH), lambda i: (0, 0, 0)),
        ],
        out_shape=[
            jax.ShapeDtypeStruct((NB * 8, 250), jnp.int32),
            jax.ShapeDtypeStruct((NSC, A * A, NH), jnp.float32),
        ],
    )(x, embed)


def _edge_call(li, tbl_rows, K):
    mesh = plsc.VectorSubcoreMesh(core_axis_name="c", subcore_axis_name="s")

    scratch = [pltpu.VMEM_SHARED((NPS, NH), jnp.float32)]
    if li == 1:
        scratch += [pltpu.VMEM_SHARED((tbl_rows, NH), jnp.float32)]
    for _ in range(K):
        scratch += [pltpu.VMEM((CH,), jnp.int32),
                    pltpu.VMEM((CH,), jnp.int32),
                    pltpu.VMEM((CH,), jnp.int32),
                    pltpu.VMEM((CH,), jnp.int32),
                    pltpu.VMEM((CH,), jnp.int32),
                    pltpu.VMEM((CH, NH), jnp.float32)]
        if li == 1:
            scratch += [pltpu.VMEM((CH, NH), jnp.float32)]
    nsem = K * (5 if li == 1 else 4)
    scratch += [pltpu.SemaphoreType.DMA] * nsem

    def body(*refs):
        if li == 0:
            src_hbm, dst_hbm, typ_hbm, tbl_hbm = refs[:4]
            w_hbm = None
            agg_out = refs[4]
            rest = list(refs[5:])
        else:
            src_hbm, dst_hbm, typ_hbm, tbl_hbm, w_hbm = refs[:5]
            agg_out = refs[5]
            rest = list(refs[6:])
        agg_sp = rest.pop(0)
        tbl_sp = rest.pop(0) if li == 1 else None
        per = 7 if li == 1 else 6
        slots = [rest[k * per:(k + 1) * per] for k in range(K)]
        sems = rest[K * per:]
        sem_it = iter(sems)
        slot_sems = [[next(sem_it) for _ in range(5 if li == 1 else 4)]
                     for _ in range(K)]

        c = lax.axis_index("c")
        s = lax.axis_index("s")

        if li == 1:
            @pl.when(s == 0)
            def _():
                pltpu.sync_copy(tbl_hbm.at[pl.ds(c * tbl_rows, tbl_rows)],
                                tbl_sp)

        zsrc = slots[0][5]

        def _zfill(r, _):
            zsrc[r, pl.ds(0, 16)] = jnp.zeros((16,), jnp.float32)
            zsrc[r, pl.ds(16, 16)] = jnp.zeros((16,), jnp.float32)
            return _
        lax.fori_loop(0, CH, _zfill, None)

        def _zcopy(j, _):
            pltpu.sync_copy(zsrc, agg_sp.at[pl.ds(s * TROWS + j * CH, CH)])
            return _
        lax.fori_loop(0, TROWS // CH, _zcopy, None)
        pltpu.sync_copy(zsrc.at[pl.ds(0, TROWS % CH)],
                        agg_sp.at[pl.ds(s * TROWS + TROWS - TROWS % CH,
                                        TROWS % CH)])
        plsc.subcore_barrier()

        base = s * 390 + jnp.minimum(s, 10)
        n = jnp.where(s < 10, 391, 390)

        def _iter(i4, _):
            i0 = i4 * K
            valid = [i0 + j < n for j in range(K)]
            handles = [[None] * 3 for _ in range(K)]

            for j in range(K):
                sv, dv = slots[j][0], slots[j][1]
                ls, ld2 = slot_sems[j][0], slot_sems[j][1]
                off = (base + i0 + j) * CH

                @pl.when(valid[j])
                def _(j=j, sv=sv, dv=dv, ls=ls, ld2=ld2, off=off):
                    handles[j][0] = pltpu.async_copy(
                        src_hbm.at[pl.ds(off, CH)], sv, ls)
                    handles[j][1] = pltpu.async_copy(
                        dst_hbm.at[pl.ds(off, CH)], dv, ld2)

            th = [[None, None] for _ in range(K)]
            for j in range(K):
                sv, dv, ts, td = slots[j][:4]
                s2, s3 = slot_sems[j][2], slot_sems[j][3]

                @pl.when(valid[j])
                def _(j=j, sv=sv, dv=dv, ts=ts, td=td, s2=s2, s3=s3):
                    handles[j][0].wait()
                    handles[j][1].wait()
                    if li == 0:
                        th[j][0] = pltpu.async_copy(typ_hbm.at[sv], ts, s2)
                    th[j][1] = pltpu.async_copy(typ_hbm.at[dv], td, s3)

            gh = [[None, None] for _ in range(K)]
            for j in range(K):
                sv, dv, ts, td, gi, msg = slots[j][:6]
                s2, s3 = slot_sems[j][2], slot_sems[j][3]

                @pl.when(valid[j])
                def _(j=j, sv=sv, ts=ts, td=td, gi=gi, msg=msg, s2=s2, s3=s3):
                    if li == 0:
                        th[j][0].wait()
                        th[j][1].wait()
                        coff = jnp.broadcast_to(c * tbl_rows,
                                                (16,)).astype(jnp.int32)
                        for g in range(CH // 16):
                            sl = pl.ds(g * 16, 16)
                            gi[sl] = ts[sl] * A + td[sl] + coff
                        gh[j][0] = pltpu.async_copy(tbl_hbm.at[gi], msg, s2)
                    else:
                        th[j][1].wait()
                        coff = jnp.broadcast_to(c * N, (16,)).astype(jnp.int32)
                        for g in range(CH // 16):
                            sl = pl.ds(g * 16, 16)
                            gi[sl] = sv[sl] + coff
                        gh[j][0] = pltpu.async_copy(w_hbm.at[gi], slots[j][6], s2)
                        gh[j][1] = pltpu.async_copy(tbl_sp.at[td], msg, s3)

            for j in range(K):
                dv, msg = slots[j][1], slots[j][5]

                @pl.when(valid[j])
                def _(j=j, dv=dv, msg=msg):
                    gh[j][0].wait()
                    if li == 1:
                        gh[j][1].wait()
                        wr = slots[j][6]

                        @plsc.parallel_loop(0, CH, 1, unroll=8)
                        def _relu_row(r):
                            for g in range(2):
                                sl = pl.ds(g * 16, 16)
                                msg[r, sl] = jnp.maximum(
                                    wr[r, sl] + msg[r, sl], 0.0)
                    pltpu.sync_copy(msg, agg_sp.at[dv], add=True)
            return _

        lax.fori_loop(0, (391 + K - 1) // K, _iter, None)
        plsc.subcore_barrier()

        def _drain(j, _):
            r = s * TROWS + j * ZROWS
            pltpu.sync_copy(agg_sp.at[pl.ds(r, ZROWS)],
                            agg_out.at[pl.ds(c * NPS + r, ZROWS)])
            return _
        lax.fori_loop(0, TROWS // ZROWS, _drain, None)

    out_type = jax.ShapeDtypeStruct((NSC * NPS, NH), jnp.float32)
    return functools.partial(
        pl.kernel, body, out_type=out_type, mesh=mesh, scratch_types=scratch,
        compiler_params=pltpu.CompilerParams(use_tc_tiling_on_sc=False))()


def _mlp(z, li, cw1, cb1, g1, b1, cw2, cb2, g2, b2):
    s1 = g1[li] * _BN_SCALE
    z = jnp.dot(z, cw1[li] * s1[None, :], precision=lax.Precision.HIGHEST)
    z = jnp.maximum(z + (cb1[li] * s1 + b1[li])[None, :], 0.0)
    s2 = g2[li] * _BN_SCALE
    z = jnp.dot(z, cw2[li] * s2[None, :], precision=lax.Precision.HIGHEST)
    return jnp.maximum(z + (cb2[li] * s2 + b2[li])[None, :], 0.0)


def _onehot_embed(typ2d, emb):
    oh = (typ2d[:, :, None] == lax.broadcasted_iota(
        jnp.int32, typ2d.shape + (A,), 2))
    return jnp.dot(oh.reshape(BN_ROWS, A).astype(jnp.float32), emb,
                   precision=lax.Precision.HIGHEST)


def _c0_body(typ_ref, agg_ref, emb_ref, cw1_ref, cb1_ref, g1_ref, b1_ref,
             cw2_ref, cb2_ref, g2_ref, b2_ref, w_ref):
    vx = _onehot_embed(typ_ref[...], emb_ref[...])
    agg = agg_ref[...]
    z = vx + jnp.concatenate([agg[0], agg[1]], axis=1)
    h = _mlp(z, 0, cw1_ref[...], cb1_ref[...], g1_ref[...], b1_ref[...],
             cw2_ref[...], cb2_ref[...], g2_ref[...], b2_ref[...])
    w = h + vx
    w_ref[...] = jnp.stack([w[:, :NH], w[:, NH:]], 0)


def _stage_c0(typ2d, agg, embed, cw1, cb1, g1, b1, cw2, cb2, g2, b2):
    wfull = pl.BlockSpec((NSC, D, D), lambda i: (0, 0, 0))
    vfull = pl.BlockSpec((NSC, D), lambda i: (0, 0))
    return pl.pallas_call(
        _c0_body,
        grid=(NB,),
        in_specs=[
            pl.BlockSpec((8, 250), lambda i: (i, 0)),
            pl.BlockSpec((NSC, BN_ROWS, NH), lambda i: (0, i, 0)),
            pl.BlockSpec((A, D), lambda i: (0, 0)),
            wfull, vfull, vfull, vfull, wfull, vfull, vfull, vfull,
        ],
        out_specs=pl.BlockSpec((NSC, BN_ROWS, NH), lambda i: (0, i, 0)),
        out_shape=jax.ShapeDtypeStruct((NSC, N, NH), jnp.float32),
    )(typ2d, agg, embed, cw1, cb1, g1, b1, cw2, cb2, g2, b2)


def _c1_body(typ_ref, w_ref, agg_ref, batch_ref, emb_ref, cw1_ref, cb1_ref,
             g1_ref, b1_ref, cw2_ref, cb2_ref, g2_ref, b2_ref, l1w_ref,
             l1b_ref, l2w_ref, l2b_ref, o_ref, pool_ref):
    i = pl.program_id(0)
    vx = _onehot_embed(typ_ref[...], emb_ref[...])
    w = w_ref[...]
    h1 = jnp.concatenate([w[0], w[1]], axis=1) - vx
    agg = agg_ref[...]
    z = h1 + jnp.concatenate([agg[0], agg[1]], axis=1)
    h2 = _mlp(z, 1, cw1_ref[...], cb1_ref[...], g1_ref[...], b1_ref[...],
              cw2_ref[...], cb2_ref[...], g2_ref[...], b2_ref[...])
    bt = batch_ref[...]
    oh = (bt[:, :, None] == lax.broadcasted_iota(
        jnp.int32, bt.shape + (B,), 2)).reshape(BN_ROWS, B)
    part = lax.dot_general(oh.astype(jnp.float32), h2,
                           (((0,), (0,)), ((), ())),
                           precision=lax.Precision.HIGHEST)

    @pl.when(i == 0)
    def _():
        pool_ref[...] = jnp.zeros_like(pool_ref)

    pool_ref[...] += part

    @pl.when(i == NB - 1)
    def _():
        p = pool_ref[...]
        o = jnp.maximum(jnp.dot(p, l1w_ref[...],
                                precision=lax.Precision.HIGHEST)
                        + l1b_ref[...], 0.0)
        o_ref[...] = jnp.dot(o, l2w_ref[...],
                             precision=lax.Precision.HIGHEST) + l2b_ref[...]


def _stage_c1(typ2d, w, agg, batch2d, embed, cw1, cb1, g1, b1, cw2, cb2, g2,
              b2, l1w, l1b, l2w, l2b):
    wfull = pl.BlockSpec((NSC, D, D), lambda i: (0, 0, 0))
    vfull = pl.BlockSpec((NSC, D), lambda i: (0, 0))
    return pl.pallas_call(
        _c1_body,
        grid=(NB,),
        in_specs=[
            pl.BlockSpec((8, 250), lambda i: (i, 0)),
            pl.BlockSpec((NSC, BN_ROWS, NH), lambda i: (0, i, 0)),
            pl.BlockSpec((NSC, BN_ROWS, NH), lambda i: (0, i, 0)),
            pl.BlockSpec((8, 250), lambda i: (i, 0)),
            pl.BlockSpec((A, D), lambda i: (0, 0)),
            wfull, vfull, vfull, vfull, wfull, vfull, vfull, vfull,
            pl.BlockSpec((D, D), lambda i: (0, 0)),
            pl.BlockSpec((1, D), lambda i: (0, 0)),
            pl.BlockSpec((D, OUT), lambda i: (0, 0)),
            pl.BlockSpec((1, OUT), lambda i: (0, 0)),
        ],
        out_specs=pl.BlockSpec((B, OUT), lambda i: (0, 0)),
        out_shape=jax.ShapeDtypeStruct((B, OUT), jnp.float32),
        scratch_shapes=[pltpu.VMEM((B, D), jnp.float32)],
    )(typ2d, w, agg, batch2d, embed, cw1, cb1, g1, b1, cw2, cb2, g2, b2,
      l1w, l1b, l2w, l2b)


def kernel(x, edge_index, batch, embed, conv_w1, conv_b1, bn1_g, bn1_b,
           conv_w2, conv_b2, bn2_g, bn2_b, lin1_w, lin1_b, lin2_w, lin2_b):
    typ2d, r0 = _stage_a(x, embed)
    typ = typ2d.reshape(N)
    src = edge_index[0]
    dst = edge_index[1]

    agg0 = _edge_call(0, A * A, 4)(src, dst, typ, r0.reshape(NSC * A * A, NH))
    w1 = _stage_c0(typ2d, agg0.reshape(NSC, NPS, NH), embed, conv_w1, conv_b1,
                   bn1_g, bn1_b, conv_w2, conv_b2, bn2_g, bn2_b)

    zpad = jnp.zeros((4, NH), jnp.float32)
    vt = jnp.concatenate([embed[:, :NH], zpad, embed[:, NH:], zpad], axis=0)
    agg1 = _edge_call(1, 104, 3)(src, dst, typ, vt, w1.reshape(NSC * N, NH))

    return _stage_c1(typ2d, w1, agg1.reshape(NSC, NPS, NH),
                     batch.reshape(NB * 8, 250), embed, conv_w1, conv_b1,
                     bn1_g, bn1_b, conv_w2, conv_b2, bn2_g, bn2_b,
                     lin1_w.reshape(D, D), lin1_b.reshape(1, D),
                     lin2_w.reshape(D, OUT), lin2_b.reshape(1, OUT))


# speedup vs baseline: 10.6433x; 1.2810x over previous
import functools

import jax
import jax.numpy as jnp
from jax import lax
from jax.experimental import pallas as pl
from jax.experimental.pallas import tpu as pltpu
from jax.experimental.pallas import tpu_sc as plsc

N = 50000
E = 800000
A = 100
D = 64
B = 128
OUT = 10

NH = 32
NSC = 2
NTILE = 16
CH = 128
NCH = E // CH
NB = 25
BN_ROWS = N // NB
NPS = 50048
TROWS = NPS // NTILE
ZROWS = 136
_BN_SCALE = 1.0 / (1.0 + 1e-5) ** 0.5


def _argmax_rows(xb):
    mx = jnp.max(xb, axis=-1, keepdims=True)
    ii = lax.broadcasted_iota(jnp.int32, xb.shape, xb.ndim - 1)
    return jnp.min(jnp.where(xb == mx, ii, A), axis=-1).astype(jnp.int32)


def _stage_a_body(x_ref, emb_ref, typ_ref, r0_ref):
    i = pl.program_id(0)
    typ_ref[...] = _argmax_rows(x_ref[...].reshape(8, 250, A))

    @pl.when(i == 0)
    def _():
        e = emb_ref[...]
        m = jnp.maximum(2.0 * e[:, None, :] + e[None, :, :], 0.0)
        r0_ref[...] = jnp.stack(
            [m[:, :, :NH].reshape(A * A, NH), m[:, :, NH:].reshape(A * A, NH)], 0)


def _stage_a(x, embed):
    return pl.pallas_call(
        _stage_a_body,
        grid=(NB,),
        in_specs=[
            pl.BlockSpec((BN_ROWS, A), lambda i: (i, 0)),
            pl.BlockSpec((A, D), lambda i: (0, 0)),
        ],
        out_specs=[
            pl.BlockSpec((8, 250), lambda i: (i, 0)),
            pl.BlockSpec((NSC, A * A, NH), lambda i: (0, 0, 0)),
        ],
        out_shape=[
            jax.ShapeDtypeStruct((NB * 8, 250), jnp.int32),
            jax.ShapeDtypeStruct((NSC, A * A, NH), jnp.float32),
        ],
    )(x, embed)


def _edge_call(li, tbl_rows, K):
    mesh = plsc.VectorSubcoreMesh(core_axis_name="c", subcore_axis_name="s")

    scratch = [pltpu.VMEM_SHARED((NPS, NH), jnp.float32)]
    if li == 0:
        scratch += [pltpu.VMEM_SHARED((N,), jnp.int32)]
    else:
        scratch += [pltpu.VMEM_SHARED((tbl_rows, NH), jnp.float32)]
    for _ in range(K):
        scratch += [pltpu.VMEM((CH,), jnp.int32),
                    pltpu.VMEM((CH,), jnp.int32),
                    pltpu.VMEM((CH,), jnp.int32),
                    pltpu.VMEM((CH,), jnp.int32),
                    pltpu.VMEM((CH,), jnp.int32),
                    pltpu.VMEM((CH, NH), jnp.float32)]
        if li == 1:
            scratch += [pltpu.VMEM((CH, NH), jnp.float32)]
    per = 7 if li == 1 else 6
    nsem = K * 5
    scratch += [pltpu.SemaphoreType.DMA] * nsem

    def body(*refs):
        if li == 0:
            src_hbm, dst_hbm, typ_hbm, tbl_hbm = refs[:4]
            w_hbm = None
            agg_out, td_out = refs[4], refs[5]
            rest = list(refs[6:])
        else:
            src_hbm, dst_hbm, tdarr_hbm, tbl_hbm, w_hbm = refs[:5]
            agg_out = refs[5]
            rest = list(refs[6:])
        agg_sp = rest.pop(0)
        aux_sp = rest.pop(0)
        slots = [rest[k * per:(k + 1) * per] for k in range(K)]
        sems = rest[K * per:]
        slot_sems = [sems[k * 5:(k + 1) * 5] for k in range(K)]

        c = lax.axis_index("c")
        s = lax.axis_index("s")

        @pl.when(s == 0)
        def _():
            if li == 0:
                pltpu.sync_copy(typ_hbm, aux_sp)
            else:
                pltpu.sync_copy(tbl_hbm.at[pl.ds(c * tbl_rows, tbl_rows)],
                                aux_sp)

        zsrc = slots[0][5]

        def _zfill(r, _):
            zsrc[r, pl.ds(0, 16)] = jnp.zeros((16,), jnp.float32)
            zsrc[r, pl.ds(16, 16)] = jnp.zeros((16,), jnp.float32)
            return _
        lax.fori_loop(0, CH, _zfill, None)

        def _zcopy(j, _):
            pltpu.sync_copy(zsrc, agg_sp.at[pl.ds(s * TROWS + j * CH, CH)])
            return _
        lax.fori_loop(0, TROWS // CH, _zcopy, None)
        pltpu.sync_copy(zsrc.at[pl.ds(0, TROWS % CH)],
                        agg_sp.at[pl.ds(s * TROWS + TROWS - TROWS % CH,
                                        TROWS % CH)])
        plsc.subcore_barrier()

        base = s * 390 + jnp.minimum(s, 10)
        n = jnp.where(s < 10, 391, 390)

        def _iter(i4, _):
            i0 = i4 * K
            valid = [i0 + j < n for j in range(K)]
            lh = [[None] * 3 for _ in range(K)]

            for j in range(K):
                sv, dv, td = slots[j][0], slots[j][1], slots[j][2]
                sm = slot_sems[j]
                off = (base + i0 + j) * CH

                @pl.when(valid[j])
                def _(j=j, sv=sv, dv=dv, td=td, sm=sm, off=off):
                    lh[j][0] = pltpu.async_copy(
                        src_hbm.at[pl.ds(off, CH)], sv, sm[0])
                    lh[j][1] = pltpu.async_copy(
                        dst_hbm.at[pl.ds(off, CH)], dv, sm[1])
                    if li == 1:
                        lh[j][2] = pltpu.async_copy(
                            tdarr_hbm.at[pl.ds(off, CH)], td, sm[2])

            th = [[None, None] for _ in range(K)]
            if li == 0:
                for j in range(K):
                    sv, dv, ts, td = slots[j][:4]
                    sm = slot_sems[j]

                    @pl.when(valid[j])
                    def _(j=j, sv=sv, dv=dv, ts=ts, td=td, sm=sm):
                        lh[j][0].wait()
                        lh[j][1].wait()
                        th[j][0] = pltpu.async_copy(aux_sp.at[sv], ts, sm[2])
                        th[j][1] = pltpu.async_copy(aux_sp.at[dv], td, sm[3])

            gh = [[None, None] for _ in range(K)]
            for j in range(K):
                sv, dv, ts, td, gi, msg = slots[j][:6]
                sm = slot_sems[j]
                off = (base + i0 + j) * CH

                @pl.when(valid[j])
                def _(j=j, sv=sv, ts=ts, td=td, gi=gi, msg=msg, sm=sm,
                      off=off):
                    if li == 0:
                        th[j][0].wait()
                        th[j][1].wait()
                        coff = jnp.broadcast_to(c * tbl_rows,
                                                (16,)).astype(jnp.int32)
                        for g in range(CH // 16):
                            sl = pl.ds(g * 16, 16)
                            gi[sl] = ts[sl] * A + td[sl] + coff
                        gh[j][0] = pltpu.async_copy(tbl_hbm.at[gi], msg, sm[4])
                    else:
                        lh[j][0].wait()
                        lh[j][1].wait()
                        lh[j][2].wait()
                        coff = jnp.broadcast_to(c * N, (16,)).astype(jnp.int32)
                        for g in range(CH // 16):
                            sl = pl.ds(g * 16, 16)
                            gi[sl] = sv[sl] + coff
                        gh[j][0] = pltpu.async_copy(w_hbm.at[gi],
                                                    slots[j][6], sm[3])
                        gh[j][1] = pltpu.async_copy(aux_sp.at[slots[j][2]],
                                                    msg, sm[4])

            if li == 0:
                for j in range(K):
                    td = slots[j][3]
                    off = (base + i0 + j) * CH

                    @pl.when(valid[j] & (c == 0))
                    def _(j=j, td=td, off=off):
                        pltpu.sync_copy(td, td_out.at[pl.ds(off, CH)])

            sh = [None] * K
            for j in range(K):
                dv, msg = slots[j][1], slots[j][5]
                sm = slot_sems[j]

                @pl.when(valid[j])
                def _(j=j, dv=dv, msg=msg, sm=sm):
                    gh[j][0].wait()
                    if li == 1:
                        gh[j][1].wait()
                        wr = slots[j][6]

                        @plsc.parallel_loop(0, CH, 1, unroll=8)
                        def _relu_row(r):
                            for g in range(2):
                                sl = pl.ds(g * 16, 16)
                                msg[r, sl] = jnp.maximum(
                                    wr[r, sl] + msg[r, sl], 0.0)
                    sh[j] = pltpu.async_copy(msg, agg_sp.at[dv], sm[0],
                                             add=True)

            for j in range(K):
                @pl.when(valid[j])
                def _(j=j):
                    sh[j].wait()
            return _

        lax.fori_loop(0, (391 + K - 1) // K, _iter, None)
        plsc.subcore_barrier()

        def _drain(j, _):
            r = s * TROWS + j * ZROWS
            pltpu.sync_copy(agg_sp.at[pl.ds(r, ZROWS)],
                            agg_out.at[pl.ds(c * NPS + r, ZROWS)])
            return _
        lax.fori_loop(0, TROWS // ZROWS, _drain, None)

    out_type = jax.ShapeDtypeStruct((NSC * NPS, NH), jnp.float32)
    if li == 0:
        out_type = [out_type, jax.ShapeDtypeStruct((E,), jnp.int32)]
    return functools.partial(
        pl.kernel, body, out_type=out_type, mesh=mesh, scratch_types=scratch,
        compiler_params=pltpu.CompilerParams(use_tc_tiling_on_sc=False))()


def _mlp(z, li, cw1, cb1, g1, b1, cw2, cb2, g2, b2):
    inv = 1.0 / jnp.sqrt(jnp.float32(1.0 + 1e-5))
    z = jnp.dot(z, cw1[li]) + cb1[li][None, :]
    z = jnp.maximum(z * inv * g1[li][None, :] + b1[li][None, :], 0.0)
    z = jnp.dot(z, cw2[li]) + cb2[li][None, :]
    return jnp.maximum(z * inv * g2[li][None, :] + b2[li][None, :], 0.0)


def _onehot_embed(typ2d, emb):
    oh = (typ2d[:, :, None] == lax.broadcasted_iota(
        jnp.int32, typ2d.shape + (A,), 2))
    return jnp.dot(oh.reshape(BN_ROWS, A).astype(jnp.float32), emb,
                   precision=lax.Precision.HIGHEST)


def _c0_body(typ_ref, agg_ref, emb_ref, cw1_ref, cb1_ref, g1_ref, b1_ref,
             cw2_ref, cb2_ref, g2_ref, b2_ref, w_ref):
    vx = _onehot_embed(typ_ref[...], emb_ref[...])
    agg = agg_ref[...]
    z = vx + jnp.concatenate([agg[0], agg[1]], axis=1)
    h = _mlp(z, 0, cw1_ref[...], cb1_ref[...], g1_ref[...], b1_ref[...],
             cw2_ref[...], cb2_ref[...], g2_ref[...], b2_ref[...])
    w = h + vx
    w_ref[...] = jnp.stack([w[:, :NH], w[:, NH:]], 0)


def _stage_c0(typ2d, agg, embed, cw1, cb1, g1, b1, cw2, cb2, g2, b2):
    wfull = pl.BlockSpec((NSC, D, D), lambda i: (0, 0, 0))
    vfull = pl.BlockSpec((NSC, D), lambda i: (0, 0))
    return pl.pallas_call(
        _c0_body,
        grid=(NB,),
        in_specs=[
            pl.BlockSpec((8, 250), lambda i: (i, 0)),
            pl.BlockSpec((NSC, BN_ROWS, NH), lambda i: (0, i, 0)),
            pl.BlockSpec((A, D), lambda i: (0, 0)),
            wfull, vfull, vfull, vfull, wfull, vfull, vfull, vfull,
        ],
        out_specs=pl.BlockSpec((NSC, BN_ROWS, NH), lambda i: (0, i, 0)),
        out_shape=jax.ShapeDtypeStruct((NSC, N, NH), jnp.float32),
    )(typ2d, agg, embed, cw1, cb1, g1, b1, cw2, cb2, g2, b2)


def _c1_body(typ_ref, w_ref, agg_ref, batch_ref, emb_ref, cw1_ref, cb1_ref,
             g1_ref, b1_ref, cw2_ref, cb2_ref, g2_ref, b2_ref, l1w_ref,
             l1b_ref, l2w_ref, l2b_ref, o_ref, pool_ref):
    i = pl.program_id(0)
    vx = _onehot_embed(typ_ref[...], emb_ref[...])
    w = w_ref[...]
    h1 = jnp.concatenate([w[0], w[1]], axis=1) - vx
    agg = agg_ref[...]
    z = h1 + jnp.concatenate([agg[0], agg[1]], axis=1)
    h2 = _mlp(z, 1, cw1_ref[...], cb1_ref[...], g1_ref[...], b1_ref[...],
              cw2_ref[...], cb2_ref[...], g2_ref[...], b2_ref[...])
    bt = batch_ref[...]
    oh = (bt[:, :, None] == lax.broadcasted_iota(
        jnp.int32, bt.shape + (B,), 2)).reshape(BN_ROWS, B)
    part = lax.dot_general(oh.astype(jnp.float32), h2,
                           (((0,), (0,)), ((), ())),
                           precision=lax.Precision.HIGHEST)

    @pl.when(i == 0)
    def _():
        pool_ref[...] = jnp.zeros_like(pool_ref)

    pool_ref[...] += part

    @pl.when(i == NB - 1)
    def _():
        p = pool_ref[...]
        o = jnp.maximum(jnp.dot(p, l1w_ref[...]) + l1b_ref[...], 0.0)
        o_ref[...] = jnp.dot(o, l2w_ref[...]) + l2b_ref[...]


def _stage_c1(typ2d, w, agg, batch2d, embed, cw1, cb1, g1, b1, cw2, cb2, g2,
              b2, l1w, l1b, l2w, l2b):
    wfull = pl.BlockSpec((NSC, D, D), lambda i: (0, 0, 0))
    vfull = pl.BlockSpec((NSC, D), lambda i: (0, 0))
    return pl.pallas_call(
        _c1_body,
        grid=(NB,),
        in_specs=[
            pl.BlockSpec((8, 250), lambda i: (i, 0)),
            pl.BlockSpec((NSC, BN_ROWS, NH), lambda i: (0, i, 0)),
            pl.BlockSpec((NSC, BN_ROWS, NH), lambda i: (0, i, 0)),
            pl.BlockSpec((8, 250), lambda i: (i, 0)),
            pl.BlockSpec((A, D), lambda i: (0, 0)),
            wfull, vfull, vfull, vfull, wfull, vfull, vfull, vfull,
            pl.BlockSpec((D, D), lambda i: (0, 0)),
            pl.BlockSpec((1, D), lambda i: (0, 0)),
            pl.BlockSpec((D, OUT), lambda i: (0, 0)),
            pl.BlockSpec((1, OUT), lambda i: (0, 0)),
        ],
        out_specs=pl.BlockSpec((B, OUT), lambda i: (0, 0)),
        out_shape=jax.ShapeDtypeStruct((B, OUT), jnp.float32),
        scratch_shapes=[pltpu.VMEM((B, D), jnp.float32)],
    )(typ2d, w, agg, batch2d, embed, cw1, cb1, g1, b1, cw2, cb2, g2, b2,
      l1w, l1b, l2w, l2b)


def kernel(x, edge_index, batch, embed, conv_w1, conv_b1, bn1_g, bn1_b,
           conv_w2, conv_b2, bn2_g, bn2_b, lin1_w, lin1_b, lin2_w, lin2_b):
    typ2d, r0 = _stage_a(x, embed)
    typ = typ2d.reshape(N)
    src = edge_index[0]
    dst = edge_index[1]

    agg0, tdarr = _edge_call(0, A * A, 4)(src, dst, typ,
                                          r0.reshape(NSC * A * A, NH))
    w1 = _stage_c0(typ2d, agg0.reshape(NSC, NPS, NH), embed, conv_w1, conv_b1,
                   bn1_g, bn1_b, conv_w2, conv_b2, bn2_g, bn2_b)

    zpad = jnp.zeros((4, NH), jnp.float32)
    vt = jnp.concatenate([embed[:, :NH], zpad, embed[:, NH:], zpad], axis=0)
    agg1 = _edge_call(1, 104, 3)(src, dst, tdarr, vt, w1.reshape(NSC * N, NH))

    return _stage_c1(typ2d, w1, agg1.reshape(NSC, NPS, NH),
                     batch.reshape(NB * 8, 250), embed, conv_w1, conv_b1,
                     bn1_g, bn1_b, conv_w2, conv_b2, bn2_g, bn2_b,
                     lin1_w.reshape(D, D), lin1_b.reshape(1, D),
                     lin2_w.reshape(D, OUT), lin2_b.reshape(1, OUT))


# speedup vs baseline: 11.3408x; 1.0655x over previous
import functools

import jax
import jax.numpy as jnp
from jax import lax
from jax.experimental import pallas as pl
from jax.experimental.pallas import tpu as pltpu
from jax.experimental.pallas import tpu_sc as plsc

N = 50000
E = 800000
A = 100
D = 64
B = 128
OUT = 10

NH = 32
NSC = 2
NTILE = 16
CH = 128
NCH = E // CH
NB = 25
BN_ROWS = N // NB
NPS = 50048
TROWS = NPS // NTILE
ZROWS = 136
_BN_SCALE = 1.0 / (1.0 + 1e-5) ** 0.5


def _argmax_rows(xb):
    mx = jnp.max(xb, axis=-1, keepdims=True)
    ii = lax.broadcasted_iota(jnp.int32, xb.shape, xb.ndim - 1)
    return jnp.min(jnp.where(xb == mx, ii, A), axis=-1).astype(jnp.int32)


def _stage_a_body(x_ref, emb_ref, typ_ref, r0_ref):
    i = pl.program_id(0)
    typ_ref[...] = _argmax_rows(x_ref[...].reshape(8, 250, A))

    @pl.when(i == 0)
    def _():
        e = emb_ref[...]
        m = jnp.maximum(2.0 * e[:, None, :] + e[None, :, :], 0.0)
        r0_ref[...] = jnp.stack(
            [m[:, :, :NH].reshape(A * A, NH), m[:, :, NH:].reshape(A * A, NH)], 0)


def _stage_a(x, embed):
    return pl.pallas_call(
        _stage_a_body,
        grid=(NB,),
        in_specs=[
            pl.BlockSpec((BN_ROWS, A), lambda i: (i, 0)),
            pl.BlockSpec((A, D), lambda i: (0, 0)),
        ],
        out_specs=[
            pl.BlockSpec((8, 250), lambda i: (i, 0)),
            pl.BlockSpec((NSC, A * A, NH), lambda i: (0, 0, 0)),
        ],
        out_shape=[
            jax.ShapeDtypeStruct((NB * 8, 250), jnp.int32),
            jax.ShapeDtypeStruct((NSC, A * A, NH), jnp.float32),
        ],
    )(x, embed)


def _edge_call(li, tbl_rows, K):
    mesh = plsc.VectorSubcoreMesh(core_axis_name="c", subcore_axis_name="s")

    scratch = [pltpu.VMEM_SHARED((NPS, NH), jnp.float32)]
    if li == 0:
        scratch += [pltpu.VMEM_SHARED((N,), jnp.int32)]
    else:
        scratch += [pltpu.VMEM_SHARED((tbl_rows, NH), jnp.float32)]
    for _ in range(K):
        scratch += [pltpu.VMEM((CH,), jnp.int32),
                    pltpu.VMEM((CH,), jnp.int32),
                    pltpu.VMEM((CH,), jnp.int32),
                    pltpu.VMEM((CH,), jnp.int32),
                    pltpu.VMEM((CH,), jnp.int32),
                    pltpu.VMEM((CH, NH), jnp.float32)]
        if li == 1:
            scratch += [pltpu.VMEM((CH, NH), jnp.float32)]
    per = 7 if li == 1 else 6
    nsem = K * 5
    scratch += [pltpu.SemaphoreType.DMA] * nsem

    def body(*refs):
        if li == 0:
            src_hbm, dst_hbm, typ_hbm, tbl_hbm = refs[:4]
            w_hbm = None
            agg_out, td_out = refs[4], refs[5]
            rest = list(refs[6:])
        else:
            src_hbm, dst_hbm, tdarr_hbm, tbl_hbm, w_hbm = refs[:5]
            agg_out = refs[5]
            rest = list(refs[6:])
        agg_sp = rest.pop(0)
        aux_sp = rest.pop(0)
        slots = [rest[k * per:(k + 1) * per] for k in range(K)]
        sems = rest[K * per:]
        slot_sems = [sems[k * 5:(k + 1) * 5] for k in range(K)]

        c = lax.axis_index("c")
        s = lax.axis_index("s")

        @pl.when(s == 0)
        def _():
            if li == 0:
                pltpu.sync_copy(typ_hbm, aux_sp)
            else:
                pltpu.sync_copy(tbl_hbm.at[pl.ds(c * tbl_rows, tbl_rows)],
                                aux_sp)

        zsrc = slots[0][5]

        def _zfill(r, _):
            zsrc[r, pl.ds(0, 16)] = jnp.zeros((16,), jnp.float32)
            zsrc[r, pl.ds(16, 16)] = jnp.zeros((16,), jnp.float32)
            return _
        lax.fori_loop(0, CH, _zfill, None)

        def _zcopy(j, _):
            pltpu.sync_copy(zsrc, agg_sp.at[pl.ds(s * TROWS + j * CH, CH)])
            return _
        lax.fori_loop(0, TROWS // CH, _zcopy, None)
        pltpu.sync_copy(zsrc.at[pl.ds(0, TROWS % CH)],
                        agg_sp.at[pl.ds(s * TROWS + TROWS - TROWS % CH,
                                        TROWS % CH)])
        plsc.subcore_barrier()

        base = s * 390 + jnp.minimum(s, 10)
        n = jnp.where(s < 10, 391, 390)

        def _iter(i4, _):
            i0 = i4 * K
            valid = [i0 + j < n for j in range(K)]
            lh = [[None] * 3 for _ in range(K)]

            for j in range(K):
                sv, dv, td = slots[j][0], slots[j][1], slots[j][2]
                sm = slot_sems[j]
                off = (base + i0 + j) * CH

                @pl.when(valid[j])
                def _(j=j, sv=sv, dv=dv, td=td, sm=sm, off=off):
                    lh[j][0] = pltpu.async_copy(
                        src_hbm.at[pl.ds(off, CH)], sv, sm[0])
                    lh[j][1] = pltpu.async_copy(
                        dst_hbm.at[pl.ds(off, CH)], dv, sm[1])
                    if li == 1:
                        lh[j][2] = pltpu.async_copy(
                            tdarr_hbm.at[pl.ds(off, CH)], td, sm[2])

            th = [[None, None] for _ in range(K)]
            if li == 0:
                for j in range(K):
                    sv, dv, ts, td = slots[j][:4]
                    sm = slot_sems[j]

                    @pl.when(valid[j])
                    def _(j=j, sv=sv, dv=dv, ts=ts, td=td, sm=sm):
                        lh[j][0].wait()
                        lh[j][1].wait()
                        th[j][0] = pltpu.async_copy(aux_sp.at[sv], ts, sm[2])
                        th[j][1] = pltpu.async_copy(aux_sp.at[dv], td, sm[3])

            gh = [[None, None] for _ in range(K)]
            for j in range(K):
                sv, dv, ts, td, gi, msg = slots[j][:6]
                sm = slot_sems[j]
                off = (base + i0 + j) * CH

                @pl.when(valid[j])
                def _(j=j, sv=sv, ts=ts, td=td, gi=gi, msg=msg, sm=sm,
                      off=off):
                    if li == 0:
                        th[j][0].wait()
                        th[j][1].wait()
                        coff = jnp.broadcast_to(c * tbl_rows,
                                                (16,)).astype(jnp.int32)
                        for g in range(CH // 16):
                            sl = pl.ds(g * 16, 16)
                            gi[sl] = ts[sl] * A + td[sl] + coff
                        gh[j][0] = pltpu.async_copy(tbl_hbm.at[gi], msg, sm[4])
                    else:
                        lh[j][0].wait()
                        lh[j][1].wait()
                        lh[j][2].wait()
                        coff = jnp.broadcast_to(c * N, (16,)).astype(jnp.int32)
                        for g in range(CH // 16):
                            sl = pl.ds(g * 16, 16)
                            gi[sl] = sv[sl] + coff
                        gh[j][0] = pltpu.async_copy(w_hbm.at[gi],
                                                    slots[j][6], sm[3])
                        gh[j][1] = pltpu.async_copy(aux_sp.at[slots[j][2]],
                                                    msg, sm[4])

            if li == 0:
                for j in range(K):
                    td = slots[j][3]
                    off = (base + i0 + j) * CH

                    @pl.when(valid[j] & (c == 0))
                    def _(j=j, td=td, off=off):
                        pltpu.sync_copy(td, td_out.at[pl.ds(off, CH)])

            sh = [None] * K
            for j in range(K):
                dv, msg = slots[j][1], slots[j][5]
                sm = slot_sems[j]

                @pl.when(valid[j])
                def _(j=j, dv=dv, msg=msg, sm=sm):
                    gh[j][0].wait()
                    if li == 1:
                        gh[j][1].wait()
                        wr = slots[j][6]

                        @plsc.parallel_loop(0, CH, 1, unroll=8)
                        def _relu_row(r):
                            for g in range(2):
                                sl = pl.ds(g * 16, 16)
                                msg[r, sl] = jnp.maximum(
                                    wr[r, sl] + msg[r, sl], 0.0)
                    sh[j] = pltpu.async_copy(msg, agg_sp.at[dv], sm[0],
                                             add=True)

            for j in range(K):
                @pl.when(valid[j])
                def _(j=j):
                    sh[j].wait()
            return _

        lax.fori_loop(0, (391 + K - 1) // K, _iter, None)
        plsc.subcore_barrier()

        def _drain(j, _):
            r = s * TROWS + j * ZROWS
            pltpu.sync_copy(agg_sp.at[pl.ds(r, ZROWS)],
                            agg_out.at[pl.ds(c * NPS + r, ZROWS)])
            return _
        lax.fori_loop(0, TROWS // ZROWS, _drain, None)

    out_type = jax.ShapeDtypeStruct((NSC * NPS, NH), jnp.float32)
    if li == 0:
        out_type = [out_type, jax.ShapeDtypeStruct((E,), jnp.int32)]
    return functools.partial(
        pl.kernel, body, out_type=out_type, mesh=mesh, scratch_types=scratch,
        compiler_params=pltpu.CompilerParams(use_tc_tiling_on_sc=False))()


def _mlp(z, li, cw1, cb1, g1, b1, cw2, cb2, g2, b2):
    inv = 1.0 / jnp.sqrt(jnp.float32(1.0 + 1e-5))
    z = jnp.dot(z, cw1[li]) + cb1[li][None, :]
    z = jnp.maximum(z * inv * g1[li][None, :] + b1[li][None, :], 0.0)
    z = jnp.dot(z, cw2[li]) + cb2[li][None, :]
    return jnp.maximum(z * inv * g2[li][None, :] + b2[li][None, :], 0.0)


def _onehot_embed(typ2d, emb):
    oh = (typ2d[:, :, None] == lax.broadcasted_iota(
        jnp.int32, typ2d.shape + (A,), 2))
    return jnp.dot(oh.reshape(BN_ROWS, A).astype(jnp.float32), emb)


def _c0_body(typ_ref, agg_ref, emb_ref, cw1_ref, cb1_ref, g1_ref, b1_ref,
             cw2_ref, cb2_ref, g2_ref, b2_ref, w_ref):
    vx = _onehot_embed(typ_ref[...], emb_ref[...])
    agg = agg_ref[...]
    z = vx + jnp.concatenate([agg[0], agg[1]], axis=1)
    h = _mlp(z, 0, cw1_ref[...], cb1_ref[...], g1_ref[...], b1_ref[...],
             cw2_ref[...], cb2_ref[...], g2_ref[...], b2_ref[...])
    w = h + vx
    w_ref[...] = jnp.stack([w[:, :NH], w[:, NH:]], 0)


def _stage_c0(typ2d, agg, embed, cw1, cb1, g1, b1, cw2, cb2, g2, b2):
    wfull = pl.BlockSpec((NSC, D, D), lambda i: (0, 0, 0))
    vfull = pl.BlockSpec((NSC, D), lambda i: (0, 0))
    return pl.pallas_call(
        _c0_body,
        grid=(NB,),
        in_specs=[
            pl.BlockSpec((8, 250), lambda i: (i, 0)),
            pl.BlockSpec((NSC, BN_ROWS, NH), lambda i: (0, i, 0)),
            pl.BlockSpec((A, D), lambda i: (0, 0)),
            wfull, vfull, vfull, vfull, wfull, vfull, vfull, vfull,
        ],
        out_specs=pl.BlockSpec((NSC, BN_ROWS, NH), lambda i: (0, i, 0)),
        out_shape=jax.ShapeDtypeStruct((NSC, N, NH), jnp.float32),
    )(typ2d, agg, embed, cw1, cb1, g1, b1, cw2, cb2, g2, b2)


def _c1_body(typ_ref, w_ref, agg_ref, batch_ref, emb_ref, cw1_ref, cb1_ref,
             g1_ref, b1_ref, cw2_ref, cb2_ref, g2_ref, b2_ref, l1w_ref,
             l1b_ref, l2w_ref, l2b_ref, o_ref, pool_ref):
    i = pl.program_id(0)
    vx = _onehot_embed(typ_ref[...], emb_ref[...])
    w = w_ref[...]
    h1 = jnp.concatenate([w[0], w[1]], axis=1) - vx
    agg = agg_ref[...]
    z = h1 + jnp.concatenate([agg[0], agg[1]], axis=1)
    h2 = _mlp(z, 1, cw1_ref[...], cb1_ref[...], g1_ref[...], b1_ref[...],
              cw2_ref[...], cb2_ref[...], g2_ref[...], b2_ref[...])
    bt = batch_ref[...]
    oh = (bt[:, :, None] == lax.broadcasted_iota(
        jnp.int32, bt.shape + (B,), 2)).reshape(BN_ROWS, B)
    part = lax.dot_general(oh.astype(jnp.float32), h2,
                           (((0,), (0,)), ((), ())))

    @pl.when(i == 0)
    def _():
        pool_ref[...] = jnp.zeros_like(pool_ref)

    pool_ref[...] += part

    @pl.when(i == NB - 1)
    def _():
        p = pool_ref[...]
        o = jnp.maximum(jnp.dot(p, l1w_ref[...]) + l1b_ref[...], 0.0)
        o_ref[...] = jnp.dot(o, l2w_ref[...]) + l2b_ref[...]


def _stage_c1(typ2d, w, agg, batch2d, embed, cw1, cb1, g1, b1, cw2, cb2, g2,
              b2, l1w, l1b, l2w, l2b):
    wfull = pl.BlockSpec((NSC, D, D), lambda i: (0, 0, 0))
    vfull = pl.BlockSpec((NSC, D), lambda i: (0, 0))
    return pl.pallas_call(
        _c1_body,
        grid=(NB,),
        in_specs=[
            pl.BlockSpec((8, 250), lambda i: (i, 0)),
            pl.BlockSpec((NSC, BN_ROWS, NH), lambda i: (0, i, 0)),
            pl.BlockSpec((NSC, BN_ROWS, NH), lambda i: (0, i, 0)),
            pl.BlockSpec((8, 250), lambda i: (i, 0)),
            pl.BlockSpec((A, D), lambda i: (0, 0)),
            wfull, vfull, vfull, vfull, wfull, vfull, vfull, vfull,
            pl.BlockSpec((D, D), lambda i: (0, 0)),
            pl.BlockSpec((1, D), lambda i: (0, 0)),
            pl.BlockSpec((D, OUT), lambda i: (0, 0)),
            pl.BlockSpec((1, OUT), lambda i: (0, 0)),
        ],
        out_specs=pl.BlockSpec((B, OUT), lambda i: (0, 0)),
        out_shape=jax.ShapeDtypeStruct((B, OUT), jnp.float32),
        scratch_shapes=[pltpu.VMEM((B, D), jnp.float32)],
    )(typ2d, w, agg, batch2d, embed, cw1, cb1, g1, b1, cw2, cb2, g2, b2,
      l1w, l1b, l2w, l2b)


def kernel(x, edge_index, batch, embed, conv_w1, conv_b1, bn1_g, bn1_b,
           conv_w2, conv_b2, bn2_g, bn2_b, lin1_w, lin1_b, lin2_w, lin2_b):
    typ2d, r0 = _stage_a(x, embed)
    typ = typ2d.reshape(N)
    src = edge_index[0]
    dst = edge_index[1]

    agg0, tdarr = _edge_call(0, A * A, 4)(src, dst, typ,
                                          r0.reshape(NSC * A * A, NH))
    w1 = _stage_c0(typ2d, agg0.reshape(NSC, NPS, NH), embed, conv_w1, conv_b1,
                   bn1_g, bn1_b, conv_w2, conv_b2, bn2_g, bn2_b)

    zpad = jnp.zeros((4, NH), jnp.float32)
    vt = jnp.concatenate([embed[:, :NH], zpad, embed[:, NH:], zpad], axis=0)
    agg1 = _edge_call(1, 104, 3)(src, dst, tdarr, vt, w1.reshape(NSC * N, NH))

    return _stage_c1(typ2d, w1, agg1.reshape(NSC, NPS, NH),
                     batch.reshape(NB * 8, 250), embed, conv_w1, conv_b1,
                     bn1_g, bn1_b, conv_w2, conv_b2, bn2_g, bn2_b,
                     lin1_w.reshape(D, D), lin1_b.reshape(1, D),
                     lin2_w.reshape(D, OUT), lin2_b.reshape(1, OUT))


# speedup vs baseline: 11.5472x; 1.0182x over previous
import functools

import jax
import jax.numpy as jnp
from jax import lax
from jax.experimental import pallas as pl
from jax.experimental.pallas import tpu as pltpu
from jax.experimental.pallas import tpu_sc as plsc

N = 50000
E = 800000
A = 100
D = 64
B = 128
OUT = 10

NH = 32
NSC = 2
NTILE = 16
CH = 128
NCH = E // CH
NB = 25
BN_ROWS = N // NB
NPS = 50048
TROWS = NPS // NTILE
ZROWS = 136
_BN_SCALE = 1.0 / (1.0 + 1e-5) ** 0.5


def _argmax_rows(xb):
    mx = jnp.max(xb, axis=-1, keepdims=True)
    ii = lax.broadcasted_iota(jnp.int32, xb.shape, xb.ndim - 1)
    return jnp.min(jnp.where(xb == mx, ii, A), axis=-1).astype(jnp.int32)


def _stage_a_body(x_ref, emb_ref, typ_ref, r0_ref):
    i = pl.program_id(0)
    typ_ref[...] = _argmax_rows(x_ref[...].reshape(8, 250, A))

    @pl.when(i == 0)
    def _():
        e = emb_ref[...]
        m = jnp.maximum(2.0 * e[:, None, :] + e[None, :, :], 0.0)
        r0_ref[...] = jnp.stack(
            [m[:, :, :NH].reshape(A * A, NH), m[:, :, NH:].reshape(A * A, NH)], 0)


def _stage_a(x, embed):
    return pl.pallas_call(
        _stage_a_body,
        grid=(NB,),
        in_specs=[
            pl.BlockSpec((BN_ROWS, A), lambda i: (i, 0)),
            pl.BlockSpec((A, D), lambda i: (0, 0)),
        ],
        out_specs=[
            pl.BlockSpec((8, 250), lambda i: (i, 0)),
            pl.BlockSpec((NSC, A * A, NH), lambda i: (0, 0, 0)),
        ],
        out_shape=[
            jax.ShapeDtypeStruct((NB * 8, 250), jnp.int32),
            jax.ShapeDtypeStruct((NSC, A * A, NH), jnp.float32),
        ],
    )(x, embed)


def _edge_call(li, tbl_rows, K):
    mesh = plsc.VectorSubcoreMesh(core_axis_name="c", subcore_axis_name="s")

    scratch = [pltpu.VMEM_SHARED((NPS, NH), jnp.float32)]
    if li == 0:
        scratch += [pltpu.VMEM_SHARED((N,), jnp.int32)]
    else:
        scratch += [pltpu.VMEM_SHARED((tbl_rows, NH), jnp.float32)]
    for _ in range(K):
        scratch += [pltpu.VMEM((CH,), jnp.int32),
                    pltpu.VMEM((CH,), jnp.int32),
                    pltpu.VMEM((CH,), jnp.int32),
                    pltpu.VMEM((CH,), jnp.int32),
                    pltpu.VMEM((CH,), jnp.int32),
                    pltpu.VMEM((CH, NH), jnp.float32)]
        if li == 1:
            scratch += [pltpu.VMEM((CH, NH), jnp.float32)]
    per = 7 if li == 1 else 6
    nsem = K * 5
    scratch += [pltpu.SemaphoreType.DMA] * nsem

    def body(*refs):
        if li == 0:
            src_hbm, dst_hbm, typ_hbm, tbl_hbm = refs[:4]
            w_hbm = None
            agg_out, td_out = refs[4], refs[5]
            rest = list(refs[6:])
        else:
            src_hbm, dst_hbm, tdarr_hbm, tbl_hbm, w_hbm = refs[:5]
            agg_out = refs[5]
            rest = list(refs[6:])
        agg_sp = rest.pop(0)
        aux_sp = rest.pop(0)
        slots = [rest[k * per:(k + 1) * per] for k in range(K)]
        sems = rest[K * per:]
        slot_sems = [sems[k * 5:(k + 1) * 5] for k in range(K)]

        c = lax.axis_index("c")
        s = lax.axis_index("s")

        @pl.when(s == 0)
        def _():
            if li == 0:
                pltpu.sync_copy(typ_hbm, aux_sp)
            else:
                pltpu.sync_copy(tbl_hbm.at[pl.ds(c * tbl_rows, tbl_rows)],
                                aux_sp)

        zsrc = slots[0][5]

        def _zfill(r, _):
            zsrc[r, pl.ds(0, 16)] = jnp.zeros((16,), jnp.float32)
            zsrc[r, pl.ds(16, 16)] = jnp.zeros((16,), jnp.float32)
            return _
        lax.fori_loop(0, CH, _zfill, None)

        def _zcopy(j, _):
            pltpu.sync_copy(zsrc, agg_sp.at[pl.ds(s * TROWS + j * CH, CH)])
            return _
        lax.fori_loop(0, TROWS // CH, _zcopy, None)
        pltpu.sync_copy(zsrc.at[pl.ds(0, TROWS % CH)],
                        agg_sp.at[pl.ds(s * TROWS + TROWS - TROWS % CH,
                                        TROWS % CH)])
        plsc.subcore_barrier()

        base = s * 390 + jnp.minimum(s, 10)
        n = jnp.where(s < 10, 391, 390)

        def _iter(i4, _):
            i0 = i4 * K
            valid = [i0 + j < n for j in range(K)]
            lh = [[None] * 3 for _ in range(K)]

            for j in range(K):
                sv, dv, td = slots[j][0], slots[j][1], slots[j][2]
                sm = slot_sems[j]
                off = (base + i0 + j) * CH

                @pl.when(valid[j])
                def _(j=j, sv=sv, dv=dv, td=td, sm=sm, off=off):
                    lh[j][0] = pltpu.async_copy(
                        src_hbm.at[pl.ds(off, CH)], sv, sm[0])
                    lh[j][1] = pltpu.async_copy(
                        dst_hbm.at[pl.ds(off, CH)], dv, sm[1])
                    if li == 1:
                        lh[j][2] = pltpu.async_copy(
                            tdarr_hbm.at[pl.ds(off, CH)], td, sm[2])

            th = [[None, None] for _ in range(K)]
            if li == 0:
                for j in range(K):
                    sv, dv, ts, td = slots[j][:4]
                    sm = slot_sems[j]

                    @pl.when(valid[j])
                    def _(j=j, sv=sv, dv=dv, ts=ts, td=td, sm=sm):
                        lh[j][0].wait()
                        lh[j][1].wait()
                        th[j][0] = pltpu.async_copy(aux_sp.at[sv], ts, sm[2])
                        th[j][1] = pltpu.async_copy(aux_sp.at[dv], td, sm[3])

            gh = [[None, None] for _ in range(K)]
            for j in range(K):
                sv, dv, ts, td, gi, msg = slots[j][:6]
                sm = slot_sems[j]
                off = (base + i0 + j) * CH

                @pl.when(valid[j])
                def _(j=j, sv=sv, ts=ts, td=td, gi=gi, msg=msg, sm=sm,
                      off=off):
                    if li == 0:
                        th[j][0].wait()
                        th[j][1].wait()
                        coff = jnp.broadcast_to(c * tbl_rows,
                                                (16,)).astype(jnp.int32)
                        for g in range(CH // 16):
                            sl = pl.ds(g * 16, 16)
                            gi[sl] = ts[sl] * A + td[sl] + coff
                        gh[j][0] = pltpu.async_copy(tbl_hbm.at[gi], msg, sm[4])
                    else:
                        lh[j][0].wait()
                        lh[j][1].wait()
                        lh[j][2].wait()
                        coff = jnp.broadcast_to(c * N, (16,)).astype(jnp.int32)
                        for g in range(CH // 16):
                            sl = pl.ds(g * 16, 16)
                            gi[sl] = sv[sl] + coff
                        gh[j][0] = pltpu.async_copy(w_hbm.at[gi],
                                                    slots[j][6], sm[3])
                        gh[j][1] = pltpu.async_copy(aux_sp.at[slots[j][2]],
                                                    msg, sm[4])

            if li == 0:
                for j in range(K):
                    td = slots[j][3]
                    off = (base + i0 + j) * CH

                    @pl.when(valid[j] & (c == 0))
                    def _(j=j, td=td, off=off):
                        pltpu.sync_copy(td, td_out.at[pl.ds(off, CH)])

            sh = [None] * K
            for j in range(K):
                dv, msg = slots[j][1], slots[j][5]
                sm = slot_sems[j]

                @pl.when(valid[j])
                def _(j=j, dv=dv, msg=msg, sm=sm):
                    gh[j][0].wait()
                    if li == 1:
                        gh[j][1].wait()
                        wr = slots[j][6]

                        @plsc.parallel_loop(0, CH, 1, unroll=8)
                        def _relu_row(r):
                            for g in range(2):
                                sl = pl.ds(g * 16, 16)
                                msg[r, sl] = jnp.maximum(
                                    wr[r, sl] + msg[r, sl], 0.0)
                    sh[j] = pltpu.async_copy(msg, agg_sp.at[dv], sm[0],
                                             add=True)

            for j in range(K):
                @pl.when(valid[j])
                def _(j=j):
                    sh[j].wait()
            return _

        lax.fori_loop(0, (391 + K - 1) // K, _iter, None)
        plsc.subcore_barrier()

        def _drain(j, _):
            r = s * TROWS + j * ZROWS
            pltpu.sync_copy(agg_sp.at[pl.ds(r, ZROWS)],
                            agg_out.at[pl.ds(c * NPS + r, ZROWS)])
            return _
        lax.fori_loop(0, TROWS // ZROWS, _drain, None)

    out_type = jax.ShapeDtypeStruct((NSC * NPS, NH), jnp.float32)
    if li == 0:
        out_type = [out_type, jax.ShapeDtypeStruct((E,), jnp.int32)]
    return functools.partial(
        pl.kernel, body, out_type=out_type, mesh=mesh, scratch_types=scratch,
        compiler_params=pltpu.CompilerParams(use_tc_tiling_on_sc=False))()


def _mlp(z, li, cw1, cb1, g1, b1, cw2, cb2, g2, b2):
    inv = 1.0 / jnp.sqrt(jnp.float32(1.0 + 1e-5))
    z = jnp.dot(z, cw1[li]) + cb1[li][None, :]
    z = jnp.maximum(z * inv * g1[li][None, :] + b1[li][None, :], 0.0)
    z = jnp.dot(z, cw2[li]) + cb2[li][None, :]
    return jnp.maximum(z * inv * g2[li][None, :] + b2[li][None, :], 0.0)


def _onehot_embed(typ2d, emb):
    oh = (typ2d[:, :, None] == lax.broadcasted_iota(
        jnp.int32, typ2d.shape + (A,), 2))
    return jnp.dot(oh.reshape(BN_ROWS, A).astype(jnp.float32), emb)


def _c0_body(typ_ref, agg_ref, emb_ref, cw1_ref, cb1_ref, g1_ref, b1_ref,
             cw2_ref, cb2_ref, g2_ref, b2_ref, w_ref):
    vx = _onehot_embed(typ_ref[...], emb_ref[...])
    agg = agg_ref[...]
    z = vx + jnp.concatenate([agg[0], agg[1]], axis=1)
    h = _mlp(z, 0, cw1_ref[...], cb1_ref[...], g1_ref[...], b1_ref[...],
             cw2_ref[...], cb2_ref[...], g2_ref[...], b2_ref[...])
    w = h + vx
    w_ref[...] = jnp.stack([w[:, :NH], w[:, NH:]], 0)


def _stage_c0(typ2d, agg, embed, cw1, cb1, g1, b1, cw2, cb2, g2, b2):
    wfull = pl.BlockSpec((NSC, D, D), lambda i: (0, 0, 0))
    vfull = pl.BlockSpec((NSC, D), lambda i: (0, 0))
    return pl.pallas_call(
        _c0_body,
        grid=(NB,),
        in_specs=[
            pl.BlockSpec((8, 250), lambda i: (i, 0)),
            pl.BlockSpec((NSC, BN_ROWS, NH), lambda i: (0, i, 0)),
            pl.BlockSpec((A, D), lambda i: (0, 0)),
            wfull, vfull, vfull, vfull, wfull, vfull, vfull, vfull,
        ],
        out_specs=pl.BlockSpec((NSC, BN_ROWS, NH), lambda i: (0, i, 0)),
        out_shape=jax.ShapeDtypeStruct((NSC, N, NH), jnp.float32),
    )(typ2d, agg, embed, cw1, cb1, g1, b1, cw2, cb2, g2, b2)


def _c1_body(typ_ref, w_ref, agg_ref, batch_ref, emb_ref, cw1_ref, cb1_ref,
             g1_ref, b1_ref, cw2_ref, cb2_ref, g2_ref, b2_ref, l1w_ref,
             l1b_ref, l2w_ref, l2b_ref, o_ref, pool_ref):
    i = pl.program_id(0)
    vx = _onehot_embed(typ_ref[...], emb_ref[...])
    w = w_ref[...]
    h1 = jnp.concatenate([w[0], w[1]], axis=1) - vx
    agg = agg_ref[...]
    z = h1 + jnp.concatenate([agg[0], agg[1]], axis=1)
    h2 = _mlp(z, 1, cw1_ref[...], cb1_ref[...], g1_ref[...], b1_ref[...],
              cw2_ref[...], cb2_ref[...], g2_ref[...], b2_ref[...])
    bt = batch_ref[...]
    oh = (bt[:, :, None] == lax.broadcasted_iota(
        jnp.int32, bt.shape + (B,), 2)).reshape(BN_ROWS, B)
    part = lax.dot_general(oh.astype(jnp.float32), h2,
                           (((0,), (0,)), ((), ())))

    @pl.when(i == 0)
    def _():
        pool_ref[...] = jnp.zeros_like(pool_ref)

    pool_ref[...] += part

    @pl.when(i == NB - 1)
    def _():
        p = pool_ref[...]
        o = jnp.maximum(jnp.dot(p, l1w_ref[...]) + l1b_ref[...], 0.0)
        o_ref[...] = jnp.dot(o, l2w_ref[...]) + l2b_ref[...]


def _stage_c1(typ2d, w, agg, batch2d, embed, cw1, cb1, g1, b1, cw2, cb2, g2,
              b2, l1w, l1b, l2w, l2b):
    wfull = pl.BlockSpec((NSC, D, D), lambda i: (0, 0, 0))
    vfull = pl.BlockSpec((NSC, D), lambda i: (0, 0))
    return pl.pallas_call(
        _c1_body,
        grid=(NB,),
        in_specs=[
            pl.BlockSpec((8, 250), lambda i: (i, 0)),
            pl.BlockSpec((NSC, BN_ROWS, NH), lambda i: (0, i, 0)),
            pl.BlockSpec((NSC, BN_ROWS, NH), lambda i: (0, i, 0)),
            pl.BlockSpec((8, 250), lambda i: (i, 0)),
            pl.BlockSpec((A, D), lambda i: (0, 0)),
            wfull, vfull, vfull, vfull, wfull, vfull, vfull, vfull,
            pl.BlockSpec((D, D), lambda i: (0, 0)),
            pl.BlockSpec((1, D), lambda i: (0, 0)),
            pl.BlockSpec((D, OUT), lambda i: (0, 0)),
            pl.BlockSpec((1, OUT), lambda i: (0, 0)),
        ],
        out_specs=pl.BlockSpec((B, OUT), lambda i: (0, 0)),
        out_shape=jax.ShapeDtypeStruct((B, OUT), jnp.float32),
        scratch_shapes=[pltpu.VMEM((B, D), jnp.float32)],
    )(typ2d, w, agg, batch2d, embed, cw1, cb1, g1, b1, cw2, cb2, g2, b2,
      l1w, l1b, l2w, l2b)


def kernel(x, edge_index, batch, embed, conv_w1, conv_b1, bn1_g, bn1_b,
           conv_w2, conv_b2, bn2_g, bn2_b, lin1_w, lin1_b, lin2_w, lin2_b):
    typ2d, r0 = _stage_a(x, embed)
    typ = typ2d.reshape(N)
    src = edge_index[0]
    dst = edge_index[1]

    agg0, tdarr = _edge_call(0, A * A, 5)(src, dst, typ,
                                          r0.reshape(NSC * A * A, NH))
    w1 = _stage_c0(typ2d, agg0.reshape(NSC, NPS, NH), embed, conv_w1, conv_b1,
                   bn1_g, bn1_b, conv_w2, conv_b2, bn2_g, bn2_b)

    zpad = jnp.zeros((4, NH), jnp.float32)
    vt = jnp.concatenate([embed[:, :NH], zpad, embed[:, NH:], zpad], axis=0)
    agg1 = _edge_call(1, 104, 3)(src, dst, tdarr, vt, w1.reshape(NSC * N, NH))

    return _stage_c1(typ2d, w1, agg1.reshape(NSC, NPS, NH),
                     batch.reshape(NB * 8, 250), embed, conv_w1, conv_b1,
                     bn1_g, bn1_b, conv_w2, conv_b2, bn2_g, bn2_b,
                     lin1_w.reshape(D, D), lin1_b.reshape(1, D),
                     lin2_w.reshape(D, OUT), lin2_b.reshape(1, OUT))
